# baseline (device time: 30314 ns/iter reference)
import jax
import jax.numpy as jnp
from jax import lax
from jax.experimental import pallas as pl
from jax.experimental.pallas import tpu as pltpu

N_DEV = 4
N_LAYERS = 3


def kernel(x, Win0, Wout0, Win1, Wout1, Win2, Wout2):
    b, d = x.shape

    def body(
        x_ref,
        win0_ref,
        wout0_ref,
        win1_ref,
        wout1_ref,
        win2_ref,
        wout2_ref,
        out_ref,
        psend_ref,
        comm_ref,
        acc_ref,
        send_sems,
        recv_sems,
    ):
        my = lax.axis_index("i")

        barrier_sem = pltpu.get_barrier_semaphore()
        for idx in range(1, N_DEV):
            peer = (my + idx) % N_DEV
            pl.semaphore_signal(
                barrier_sem,
                inc=1,
                device_id=(peer,),
                device_id_type=pl.DeviceIdType.MESH,
            )
        pl.semaphore_wait(barrier_sem, N_DEV - 1)

        wins = [win0_ref, win1_ref, win2_ref]
        wouts = [wout0_ref, wout1_ref, wout2_ref]

        xv = x_ref[:, :]
        for l in range(N_LAYERS):
            h = jnp.maximum(
                jnp.dot(xv, wins[l][:, :], preferred_element_type=jnp.float32),
                0.0,
            )
            p = jnp.dot(h, wouts[l][:, :], preferred_element_type=jnp.float32)
            psend_ref[l, :, :] = p

            sends = []
            for idx in range(1, N_DEV):
                peer = (my + idx) % N_DEV
                rdma = pltpu.make_async_remote_copy(
                    src_ref=psend_ref.at[l],
                    dst_ref=comm_ref.at[l, N_DEV - 1 - idx],
                    send_sem=send_sems.at[l, idx - 1],
                    recv_sem=recv_sems.at[l, N_DEV - 1 - idx],
                    device_id=(peer,),
                    device_id_type=pl.DeviceIdType.MESH,
                )
                rdma.start()
                sends.append(rdma)

            acc = psend_ref[l, :, :]
            for j in range(N_DEV - 1):
                recv = pltpu.make_async_remote_copy(
                    src_ref=comm_ref.at[l, j],
                    dst_ref=comm_ref.at[l, j],
                    send_sem=send_sems.at[l, j],
                    recv_sem=recv_sems.at[l, j],
                    device_id=(my,),
                    device_id_type=pl.DeviceIdType.MESH,
                )
                recv.wait_recv()
                acc = acc + comm_ref[l, j, :, :]

            for rdma in sends:
                rdma.wait_send()
            xv = acc

        acc_ref[:, :] = xv
        rows = b // N_DEV
        out_ref[:, :] = acc_ref[pl.ds(my * rows, rows), :]

    return pl.pallas_call(
        body,
        out_shape=jax.ShapeDtypeStruct((b // N_DEV, d), jnp.float32),
        in_specs=[pl.BlockSpec(memory_space=pltpu.VMEM)] * 7,
        out_specs=pl.BlockSpec(memory_space=pltpu.VMEM),
        scratch_shapes=[
            pltpu.VMEM((N_LAYERS, b, d), jnp.float32),
            pltpu.VMEM((N_LAYERS, N_DEV - 1, b, d), jnp.float32),
            pltpu.VMEM((b, d), jnp.float32),
            pltpu.SemaphoreType.DMA((N_LAYERS, N_DEV - 1)),
            pltpu.SemaphoreType.DMA((N_LAYERS, N_DEV - 1)),
        ],
        compiler_params=pltpu.CompilerParams(collective_id=0),
    )(x, Win0, Wout0, Win1, Wout1, Win2, Wout2)


# device time: 28216 ns/iter; 1.0744x vs baseline; 1.0744x over previous
import jax
import jax.numpy as jnp
from jax import lax
from jax.experimental import pallas as pl
from jax.experimental.pallas import tpu as pltpu

N_DEV = 4
N_CHUNK = 4


def kernel(x, Win0, Wout0, Win1, Wout1, Win2, Wout2):
    b, d = x.shape
    ck = d // N_CHUNK
    rows = b // N_DEV

    def body(
        x_ref,
        win0_ref,
        wout0_ref,
        win1_ref,
        wout1_ref,
        win2_ref,
        wout2_ref,
        out_ref,
        psend_ref,
        comm_ref,
        p2_ref,
        rs_ref,
        send_sems,
        recv_sems,
        rs_send_sems,
        rs_recv_sems,
    ):
        my = lax.axis_index("i")

        barrier_sem = pltpu.get_barrier_semaphore()
        for idx in range(1, N_DEV):
            pl.semaphore_signal(
                barrier_sem,
                inc=1,
                device_id=((my + idx) % N_DEV,),
                device_id_type=pl.DeviceIdType.MESH,
            )
        pl.semaphore_wait(barrier_sem, N_DEV - 1)

        wins = [win0_ref, win1_ref, win2_ref]
        wouts = [wout0_ref, wout1_ref, wout2_ref]
        sends = []

        h = jnp.maximum(
            jnp.dot(x_ref[:, :], win0_ref[:, :], preferred_element_type=jnp.float32),
            0.0,
        )

        for l in range(2):
            win_next = wins[l + 1]
            for c in range(N_CHUNK):
                pc = jnp.dot(
                    h,
                    wouts[l][:, c * ck : (c + 1) * ck],
                    preferred_element_type=jnp.float32,
                )
                psend_ref[l, c, :, :] = pc
                for idx in range(1, N_DEV):
                    rdma = pltpu.make_async_remote_copy(
                        src_ref=psend_ref.at[l, c],
                        dst_ref=comm_ref.at[l, c, N_DEV - 1 - idx],
                        send_sem=send_sems.at[l, c, idx - 1],
                        recv_sem=recv_sems.at[l, c, N_DEV - 1 - idx],
                        device_id=((my + idx) % N_DEV,),
                        device_id_type=pl.DeviceIdType.MESH,
                    )
                    rdma.start()
                    sends.append(rdma)

            acc_h = jnp.zeros((b, win_next.shape[1]), jnp.float32)
            for c in range(N_CHUNK):
                xc = psend_ref[l, c, :, :]
                for j in range(N_DEV - 1):
                    recv = pltpu.make_async_remote_copy(
                        src_ref=comm_ref.at[l, c, j],
                        dst_ref=comm_ref.at[l, c, j],
                        send_sem=send_sems.at[l, c, j],
                        recv_sem=recv_sems.at[l, c, j],
                        device_id=(my,),
                        device_id_type=pl.DeviceIdType.MESH,
                    )
                    recv.wait_recv()
                    xc = xc + comm_ref[l, c, j, :, :]
                acc_h = acc_h + jnp.dot(
                    xc,
                    win_next[c * ck : (c + 1) * ck, :],
                    preferred_element_type=jnp.float32,
                )
            h = jnp.maximum(acc_h, 0.0)

        p2_ref[:, :] = jnp.dot(
            h, wout2_ref[:, :], preferred_element_type=jnp.float32
        )
        for idx in range(1, N_DEV):
            peer = (my + idx) % N_DEV
            rdma = pltpu.make_async_remote_copy(
                src_ref=p2_ref.at[pl.ds(peer * rows, rows)],
                dst_ref=rs_ref.at[N_DEV - 1 - idx],
                send_sem=rs_send_sems.at[idx - 1],
                recv_sem=rs_recv_sems.at[N_DEV - 1 - idx],
                device_id=(peer,),
                device_id_type=pl.DeviceIdType.MESH,
            )
            rdma.start()
            sends.append(rdma)

        own = p2_ref[pl.ds(my * rows, rows), :]
        for j in range(N_DEV - 1):
            recv = pltpu.make_async_remote_copy(
                src_ref=rs_ref.at[j],
                dst_ref=rs_ref.at[j],
                send_sem=rs_send_sems.at[j],
                recv_sem=rs_recv_sems.at[j],
                device_id=(my,),
                device_id_type=pl.DeviceIdType.MESH,
            )
            recv.wait_recv()
            own = own + rs_ref[j, :, :]
        out_ref[:, :] = own

        for rdma in sends:
            rdma.wait_send()

    return pl.pallas_call(
        body,
        out_shape=jax.ShapeDtypeStruct((rows, d), jnp.float32),
        in_specs=[pl.BlockSpec(memory_space=pltpu.VMEM)] * 7,
        out_specs=pl.BlockSpec(memory_space=pltpu.VMEM),
        scratch_shapes=[
            pltpu.VMEM((2, N_CHUNK, b, ck), jnp.float32),
            pltpu.VMEM((2, N_CHUNK, N_DEV - 1, b, ck), jnp.float32),
            pltpu.VMEM((b, d), jnp.float32),
            pltpu.VMEM((N_DEV - 1, rows, d), jnp.float32),
            pltpu.SemaphoreType.DMA((2, N_CHUNK, N_DEV - 1)),
            pltpu.SemaphoreType.DMA((2, N_CHUNK, N_DEV - 1)),
            pltpu.SemaphoreType.DMA((N_DEV - 1,)),
            pltpu.SemaphoreType.DMA((N_DEV - 1,)),
        ],
        compiler_params=pltpu.CompilerParams(collective_id=0),
    )(x, Win0, Wout0, Win1, Wout1, Win2, Wout2)
